# baseline (speedup 1.0000x reference)
import functools

import jax
import jax.numpy as jnp
from jax.experimental import pallas as pl
from jax.experimental.pallas import tpu as pltpu

_T = 32768
_H = 768
_I = 2048
_L = 256
_TILE = 1024
_EPS = 1e-5


def _fold_kernel(fc1_ref, ew_ref, fc2_ref, o_ref):
    a = jax.lax.dot_general(
        fc2_ref[...], ew_ref[...], (((1,), (1,)), ((), ())),
        preferred_element_type=jnp.float32)
    o_ref[...] = jax.lax.dot_general(
        a, fc1_ref[...], (((1,), (0,)), ((), ())),
        preferred_element_type=jnp.float32)


def _fused_kernel(x_ref, w1_ref, down_ref, g_ref, b_ref, o_ref):
    x = x_ref[...].astype(jnp.bfloat16)
    h = jax.lax.dot_general(
        x, w1_ref[...], (((1,), (1,)), ((), ())),
        preferred_element_type=jnp.float32)
    s = jnp.maximum(h[:, :_I], 0.0)
    s = (s * s).astype(jnp.bfloat16)
    shared = jax.lax.dot_general(
        s, down_ref[...], (((1,), (1,)), ((), ())),
        preferred_element_type=jnp.float32)
    y = shared + h[:, _I:]
    mu = jnp.mean(y, axis=-1, keepdims=True)
    yc = y - mu
    var = jnp.mean(yc * yc, axis=-1, keepdims=True)
    o_ref[...] = yc * jax.lax.rsqrt(var + _EPS) * g_ref[...] + b_ref[...]


@functools.partial(jax.jit, static_argnames=())
def kernel(hidden_states, gate_w, up_w, down_w, fc1_w, fc2_w, expert_w, ln_g, ln_b):
    del gate_w

    wlat_t = pl.pallas_call(
        _fold_kernel,
        out_shape=jax.ShapeDtypeStruct((_H, _H), jnp.float32),
    )(fc1_w, expert_w, fc2_w)
    w1 = jnp.concatenate([up_w, wlat_t], axis=0).astype(jnp.bfloat16)
    down_bf = down_w.astype(jnp.bfloat16)

    grid = (_T // _TILE,)
    out = pl.pallas_call(
        _fused_kernel,
        grid=grid,
        in_specs=[
            pl.BlockSpec((_TILE, _H), lambda i: (i, 0)),
            pl.BlockSpec((_I + _H, _H), lambda i: (0, 0)),
            pl.BlockSpec((_H, _I), lambda i: (0, 0)),
            pl.BlockSpec((1, _H), lambda i: (0, 0)),
            pl.BlockSpec((1, _H), lambda i: (0, 0)),
        ],
        out_specs=pl.BlockSpec((_TILE, _H), lambda i: (i, 0)),
        out_shape=jax.ShapeDtypeStruct((_T, _H), jnp.float32),
        compiler_params=pltpu.CompilerParams(
            dimension_semantics=("parallel",)),
    )(hidden_states, w1, down_bf,
      ln_g.reshape(1, _H), ln_b.reshape(1, _H))
    return out


# speedup vs baseline: 1.0800x; 1.0800x over previous
import functools

import jax
import jax.numpy as jnp
from jax.experimental import pallas as pl
from jax.experimental.pallas import tpu as pltpu

_T = 32768
_H = 768
_I = 2048
_L = 256
_TILE = 1024
_EPS = 1e-5


def _prep_kernel(ew_ref, fc2_ref, o_ref):
    o_ref[...] = jax.lax.dot_general(
        fc2_ref[...], ew_ref[...], (((1,), (1,)), ((), ())),
        preferred_element_type=jnp.float32)


def _fused_kernel(x_ref, up_ref, down_ref, fc1_ref, m2_ref, g_ref, b_ref, o_ref):
    x = x_ref[...].astype(jnp.bfloat16)
    xl = jax.lax.dot_general(
        x, fc1_ref[...].astype(jnp.bfloat16), (((1,), (1,)), ((), ())),
        preferred_element_type=jnp.float32)
    h = jax.lax.dot_general(
        x, up_ref[...].astype(jnp.bfloat16), (((1,), (1,)), ((), ())),
        preferred_element_type=jnp.float32)
    lat = jax.lax.dot_general(
        xl, m2_ref[...], (((1,), (1,)), ((), ())),
        preferred_element_type=jnp.float32)
    s = jnp.maximum(h, 0.0)
    s = (s * s).astype(jnp.bfloat16)
    shared = jax.lax.dot_general(
        s, down_ref[...].astype(jnp.bfloat16), (((1,), (1,)), ((), ())),
        preferred_element_type=jnp.float32)
    y = shared + lat
    mu = jnp.mean(y, axis=-1, keepdims=True)
    yc = y - mu
    var = jnp.mean(yc * yc, axis=-1, keepdims=True)
    o_ref[...] = yc * jax.lax.rsqrt(var + _EPS) * g_ref[...] + b_ref[...]


@functools.partial(jax.jit, static_argnames=())
def kernel(hidden_states, gate_w, up_w, down_w, fc1_w, fc2_w, expert_w, ln_g, ln_b):
    del gate_w

    m2 = pl.pallas_call(
        _prep_kernel,
        out_shape=jax.ShapeDtypeStruct((_H, _L), jnp.float32),
    )(expert_w, fc2_w)

    grid = (_T // _TILE,)
    out = pl.pallas_call(
        _fused_kernel,
        grid=grid,
        in_specs=[
            pl.BlockSpec((_TILE, _H), lambda i: (i, 0)),
            pl.BlockSpec((_I, _H), lambda i: (0, 0)),
            pl.BlockSpec((_H, _I), lambda i: (0, 0)),
            pl.BlockSpec((_L, _H), lambda i: (0, 0)),
            pl.BlockSpec((_H, _L), lambda i: (0, 0)),
            pl.BlockSpec((1, _H), lambda i: (0, 0)),
            pl.BlockSpec((1, _H), lambda i: (0, 0)),
        ],
        out_specs=pl.BlockSpec((_TILE, _H), lambda i: (i, 0)),
        out_shape=jax.ShapeDtypeStruct((_T, _H), jnp.float32),
        compiler_params=pltpu.CompilerParams(
            dimension_semantics=("parallel",)),
    )(hidden_states, up_w, down_w, fc1_w, m2,
      ln_g.reshape(1, _H), ln_b.reshape(1, _H))
    return out


# speedup vs baseline: 1.1105x; 1.0283x over previous
import functools

import jax
import jax.numpy as jnp
from jax.experimental import pallas as pl
from jax.experimental.pallas import tpu as pltpu

_T = 32768
_H = 768
_I = 2048
_L = 256
_TILE = 1024
_EPS = 1e-5


def _prep_kernel(ew_ref, fc2_ref, o_ref):
    o_ref[...] = jax.lax.dot_general(
        fc2_ref[...], ew_ref[...], (((1,), (1,)), ((), ())),
        preferred_element_type=jnp.float32)


def _fused_kernel(x_ref, up_ref, down_ref, fc1_ref, m2_ref, g_ref, b_ref, o_ref):
    up = up_ref[...].astype(jnp.bfloat16)
    down = down_ref[...].astype(jnp.bfloat16)
    fc1 = fc1_ref[...].astype(jnp.bfloat16)
    m2 = m2_ref[...]
    g = g_ref[...]
    b = b_ref[...]
    half = _TILE // 4
    for k in range(4):
        x = x_ref[k * half:(k + 1) * half, :].astype(jnp.bfloat16)
        xl = jax.lax.dot_general(
            x, fc1, (((1,), (1,)), ((), ())),
            preferred_element_type=jnp.float32)
        h = jax.lax.dot_general(
            x, up, (((1,), (1,)), ((), ())),
            preferred_element_type=jnp.float32)
        lat = jax.lax.dot_general(
            xl, m2, (((1,), (1,)), ((), ())),
            preferred_element_type=jnp.float32)
        s = jnp.maximum(h, 0.0)
        s = (s * s).astype(jnp.bfloat16)
        shared = jax.lax.dot_general(
            s, down, (((1,), (1,)), ((), ())),
            preferred_element_type=jnp.float32)
        y = shared + lat
        mu = jnp.mean(y, axis=-1, keepdims=True)
        yc = y - mu
        var = jnp.mean(yc * yc, axis=-1, keepdims=True)
        o_ref[k * half:(k + 1) * half, :] = (
            yc * jax.lax.rsqrt(var + _EPS) * g + b)


@functools.partial(jax.jit, static_argnames=())
def kernel(hidden_states, gate_w, up_w, down_w, fc1_w, fc2_w, expert_w, ln_g, ln_b):
    del gate_w

    m2 = pl.pallas_call(
        _prep_kernel,
        out_shape=jax.ShapeDtypeStruct((_H, _L), jnp.float32),
    )(expert_w, fc2_w)

    grid = (_T // _TILE,)
    out = pl.pallas_call(
        _fused_kernel,
        grid=grid,
        in_specs=[
            pl.BlockSpec((_TILE, _H), lambda i: (i, 0)),
            pl.BlockSpec((_I, _H), lambda i: (0, 0)),
            pl.BlockSpec((_H, _I), lambda i: (0, 0)),
            pl.BlockSpec((_L, _H), lambda i: (0, 0)),
            pl.BlockSpec((_H, _L), lambda i: (0, 0)),
            pl.BlockSpec((1, _H), lambda i: (0, 0)),
            pl.BlockSpec((1, _H), lambda i: (0, 0)),
        ],
        out_specs=pl.BlockSpec((_TILE, _H), lambda i: (i, 0)),
        out_shape=jax.ShapeDtypeStruct((_T, _H), jnp.float32),
        compiler_params=pltpu.CompilerParams(
            dimension_semantics=("parallel",)),
    )(hidden_states, up_w, down_w, fc1_w, m2,
      ln_g.reshape(1, _H), ln_b.reshape(1, _H))
    return out


# speedup vs baseline: 1.1249x; 1.0130x over previous
import functools

import jax
import jax.numpy as jnp
from jax.experimental import pallas as pl
from jax.experimental.pallas import tpu as pltpu

_T = 32768
_H = 768
_I = 2048
_L = 256
_TILE = 2048
_EPS = 1e-5


def _prep_kernel(ew_ref, fc2_ref, up_ref, down_ref, fc1_ref,
                 m2_ref, upb_ref, dnb_ref, fc1b_ref):
    m2_ref[...] = jax.lax.dot_general(
        fc2_ref[...], ew_ref[...], (((1,), (1,)), ((), ())),
        preferred_element_type=jnp.float32).astype(jnp.bfloat16)
    upb_ref[...] = up_ref[...].astype(jnp.bfloat16)
    dnb_ref[...] = down_ref[...].astype(jnp.bfloat16)
    fc1b_ref[...] = fc1_ref[...].astype(jnp.bfloat16)


def _fused_kernel(x_ref, up_ref, down_ref, fc1_ref, m2_ref, g_ref, b_ref, o_ref):
    up = up_ref[...]
    down = down_ref[...]
    fc1 = fc1_ref[...]
    m2 = m2_ref[...]
    g = g_ref[...]
    b = b_ref[...]
    half = _TILE // 8
    for k in range(8):
        x = x_ref[k * half:(k + 1) * half, :].astype(jnp.bfloat16)
        xl = jax.lax.dot_general(
            x, fc1, (((1,), (1,)), ((), ())),
            preferred_element_type=jnp.float32)
        h = jax.lax.dot_general(
            x, up, (((1,), (1,)), ((), ())),
            preferred_element_type=jnp.float32)
        lat = jax.lax.dot_general(
            xl, m2, (((1,), (1,)), ((), ())),
            preferred_element_type=jnp.float32)
        s = jnp.maximum(h, 0.0)
        s = (s * s).astype(jnp.bfloat16)
        shared = jax.lax.dot_general(
            s, down, (((1,), (1,)), ((), ())),
            preferred_element_type=jnp.float32)
        y = shared + lat
        mu = jnp.mean(y, axis=-1, keepdims=True)
        yc = y - mu
        var = jnp.mean(yc * yc, axis=-1, keepdims=True)
        o_ref[k * half:(k + 1) * half, :] = (
            yc * jax.lax.rsqrt(var + _EPS) * g + b)


@functools.partial(jax.jit, static_argnames=())
def kernel(hidden_states, gate_w, up_w, down_w, fc1_w, fc2_w, expert_w, ln_g, ln_b):
    del gate_w

    m2, up_bf, down_bf, fc1_bf = pl.pallas_call(
        _prep_kernel,
        out_shape=(
            jax.ShapeDtypeStruct((_H, _L), jnp.bfloat16),
            jax.ShapeDtypeStruct((_I, _H), jnp.bfloat16),
            jax.ShapeDtypeStruct((_H, _I), jnp.bfloat16),
            jax.ShapeDtypeStruct((_L, _H), jnp.bfloat16),
        ),
    )(expert_w, fc2_w, up_w, down_w, fc1_w)

    grid = (_T // _TILE,)
    out = pl.pallas_call(
        _fused_kernel,
        grid=grid,
        in_specs=[
            pl.BlockSpec((_TILE, _H), lambda i: (i, 0)),
            pl.BlockSpec((_I, _H), lambda i: (0, 0)),
            pl.BlockSpec((_H, _I), lambda i: (0, 0)),
            pl.BlockSpec((_L, _H), lambda i: (0, 0)),
            pl.BlockSpec((_H, _L), lambda i: (0, 0)),
            pl.BlockSpec((1, _H), lambda i: (0, 0)),
            pl.BlockSpec((1, _H), lambda i: (0, 0)),
        ],
        out_specs=pl.BlockSpec((_TILE, _H), lambda i: (i, 0)),
        out_shape=jax.ShapeDtypeStruct((_T, _H), jnp.float32),
        compiler_params=pltpu.CompilerParams(
            dimension_semantics=("parallel",)),
    )(hidden_states, up_bf, down_bf, fc1_bf, m2,
      ln_g.reshape(1, _H), ln_b.reshape(1, _H))
    return out
